# baseline (speedup 1.0000x reference)
import functools

import jax
import jax.numpy as jnp
from jax import lax
from jax.experimental import pallas as pl
from jax.experimental.pallas import tpu as pltpu
from jax.experimental.pallas import tpu_sc as plsc

B = 2
N = 4096
DIM = 1024
TRASH_SLOT = 2048
SEL_PAD = 2056
NW = 32
NUM_HEAVY = 1024
LIGHT_H = 512
HEAVY_H = 4096
K_CD = 8
EPS_CD = 0.1
N_ITERS = 50

BT_L = 1024
N_STEPS_L = (B * N) // BT_L
BT_H = 256
N_STEPS_H = (B * NUM_HEAVY) // BT_H


def _rmsnorm(x, gamma):
    norm = jnp.sqrt(jnp.sum(x * x, axis=-1, keepdims=True))
    normed = x / jnp.maximum(norm, 1e-12)
    return normed * (DIM ** 0.5) * gamma


def _gelu(x):
    return 0.5 * x * (1.0 + lax.erf(x * (2.0 ** -0.5)))


def _cumsum_lanes(m_f32, nchunk, clen):
    rows = m_f32.reshape(B * nchunk, clen)
    ii = lax.broadcasted_iota(jnp.int32, (clen, clen), 0)
    jj = lax.broadcasted_iota(jnp.int32, (clen, clen), 1)
    tri = (ii <= jj).astype(jnp.float32)
    within = jnp.dot(rows, tri, preferred_element_type=jnp.float32)
    totals = within[:, clen - 1:clen].reshape(B, nchunk)
    ci = lax.broadcasted_iota(jnp.int32, (nchunk, nchunk), 0)
    cj = lax.broadcasted_iota(jnp.int32, (nchunk, nchunk), 1)
    stri = (ci < cj).astype(jnp.float32)
    offs = jnp.dot(totals, stri, preferred_element_type=jnp.float32)
    out = within.reshape(B, nchunk, clen) + offs[:, :, None]
    return out.reshape(B, nchunk * clen)


def _select_slots(scores):
    key = lax.bitcast_convert_type(scores, jnp.int32)
    T = jnp.zeros((B, 1), jnp.int32)
    for bit in range(30, -1, -1):
        cand = T | (1 << bit)
        cnt = jnp.sum((key >= cand).astype(jnp.int32), axis=-1, keepdims=True)
        T = jnp.where(cnt >= NUM_HEAVY, cand, T)
    gt = key > T
    cnt_gt = jnp.sum(gt.astype(jnp.int32), axis=-1, keepdims=True)
    m = NUM_HEAVY - cnt_gt
    eq = key == T
    rank_eq = _cumsum_lanes(eq.astype(jnp.float32), 32, N // 32)
    sel = gt | (eq & (rank_eq <= m.astype(jnp.float32)))
    pos = _cumsum_lanes(sel.astype(jnp.float32), 32, N // 32) - 1.0
    slot = pos.astype(jnp.int32) + (
        lax.broadcasted_iota(jnp.int32, (B, N), 0) * NUM_HEAVY)
    return jnp.where(sel, slot, TRASH_SLOT)


def _compact_matmul(slot):
    slot_flat = slot.reshape(B * N)
    nch = 16
    clen = (B * N) // nch
    acc_hi = jnp.zeros((1, B * NUM_HEAVY), jnp.float32)
    acc_lo = jnp.zeros((1, B * NUM_HEAVY), jnp.float32)
    iota_p = lax.broadcasted_iota(jnp.int32, (clen, B * NUM_HEAVY), 1)
    for c in range(nch):
        sl = slot_flat[c * clen:(c + 1) * clen]
        onehot = (sl[:, None] == iota_p).astype(jnp.float32)
        tok = lax.iota(jnp.int32, clen) + (c * clen)
        tok_hi = (tok // 64).astype(jnp.float32).reshape(1, clen)
        tok_lo = (tok % 64).astype(jnp.float32).reshape(1, clen)
        acc_hi = acc_hi + jnp.dot(tok_hi, onehot,
                                  preferred_element_type=jnp.float32)
        acc_lo = acc_lo + jnp.dot(tok_lo, onehot,
                                  preferred_element_type=jnp.float32)
    return (acc_hi * 64.0 + acc_lo).astype(jnp.int32)


def _router_kernel(x_ref, rt_ref, sel_ref, s_acc):
    j = pl.program_id(0)
    xb = x_ref[...]
    s_part = jnp.dot(xb, rt_ref[...], preferred_element_type=jnp.float32)
    s_acc[pl.ds(j, 1), :] = s_part.reshape(1, BT_L)

    @pl.when(j == N_STEPS_L - 1)
    def _():
        s = s_acc[...].reshape(B, N)
        constant = EPS_CD * jnp.log(float(K_CD))
        b = -jnp.maximum(s, 0.0)
        a = jnp.zeros((B, 1), dtype=s.dtype)

        def body(_, carry):
            a, b = carry
            z = (s + b) / EPS_CD
            m = jnp.max(z, axis=-1, keepdims=True)
            lse = jnp.log(jnp.sum(jnp.exp(z - m), axis=-1, keepdims=True)) + m
            a = constant - EPS_CD * lse
            b = -jnp.maximum(s + a, 0.0)
            return a, b

        a, b = lax.fori_loop(0, N_ITERS, body, (a, b))
        scores = jnp.exp((s + a + b) / EPS_CD)
        sel_ref[...] = _compact_matmul(_select_slots(scores))


def _light_kernel(x_ref, gl_ref, w1_ref, b1_ref, w2_ref, b2_ref, light_ref):
    xb = x_ref[...]
    h = _rmsnorm(xb, gl_ref[...])
    h = jnp.dot(h, w1_ref[...], preferred_element_type=jnp.float32) + b1_ref[...]
    h = _gelu(h)
    light_ref[...] = (
        jnp.dot(h, w2_ref[...], preferred_element_type=jnp.float32) + b2_ref[...]
    )


ROWS_PER_W = (B * NUM_HEAVY) // NW
SUB = 32

_GATHER_CACHE = []


def _get_gather_kernel():
    if _GATHER_CACHE:
        return _GATHER_CACHE[0]

    @functools.partial(
        pl.kernel,
        out_type=jax.ShapeDtypeStruct((B * NUM_HEAVY, DIM), jnp.float32),
        mesh=plsc.VectorSubcoreMesh(core_axis_name="c", subcore_axis_name="s"),
        scratch_types=[
            pltpu.VMEM((ROWS_PER_W,), jnp.int32),
            pltpu.VMEM((2, SUB, DIM), jnp.float32),
            pltpu.SemaphoreType.DMA((2,)),
        ],
    )
    def _gather_kernel(sel_hbm, x_hbm, xsel_hbm, idx_v, buf, sems):
        wid = lax.axis_index("s") * 2 + lax.axis_index("c")
        base = wid * ROWS_PER_W
        pltpu.sync_copy(sel_hbm.at[pl.ds(base, ROWS_PER_W)], idx_v)
        cps = [pltpu.async_copy(x_hbm.at[idx_v.at[pl.ds(t * SUB, SUB)]],
                                buf.at[t], sems.at[t])
               for t in range(2)]
        for t in range(2):
            cps[t].wait()
            pltpu.sync_copy(buf.at[t],
                            xsel_hbm.at[pl.ds(base + t * SUB, SUB)])

    _GATHER_CACHE.append(_gather_kernel)
    return _gather_kernel


def _heavy_kernel(sel_ref, xsel_ref, _light_alias, gh_ref, w3_ref, b3_ref,
                  w4_ref, b4_ref, out_hbm, lbuf, obuf, sem_l, sem_o):
    del _light_alias
    j = pl.program_id(0)

    def l_copy(jj, i):
        gidx = sel_ref[jj * BT_H + i]
        s = jax.lax.rem(jj, 2)
        return pltpu.make_async_copy(
            out_hbm.at[pl.ds(gidx, 1), :], lbuf.at[s, pl.ds(i, 1), :],
            sem_l.at[s])

    def o_copy(jj, i):
        gidx = sel_ref[jj * BT_H + i]
        s = jax.lax.rem(jj, 2)
        return pltpu.make_async_copy(
            obuf.at[s, pl.ds(i, 1), :], out_hbm.at[pl.ds(gidx, 1), :],
            sem_o.at[s])

    def issue_gathers(jj):
        lax.fori_loop(0, BT_H,
                      lambda i, _: (l_copy(jj, i).start(), 0)[1], 0,
                      unroll=32)

    @pl.when(j == 0)
    def _():
        issue_gathers(0)

    @pl.when(j + 1 < N_STEPS_H)
    def _():
        issue_gathers(j + 1)

    @pl.when(j >= 2)
    def _():
        lax.fori_loop(0, BT_H, lambda i, _: (o_copy(j - 2, i).wait(), 0)[1],
                      0, unroll=32)

    s = jax.lax.rem(j, 2)
    xb = xsel_ref[...]
    h = _rmsnorm(xb, gh_ref[...])
    h = jnp.dot(h, w3_ref[...], preferred_element_type=jnp.float32) + b3_ref[...]
    h = _gelu(h)
    heavy = jnp.dot(h, w4_ref[...], preferred_element_type=jnp.float32) + b4_ref[...]

    lax.fori_loop(0, BT_H, lambda i, _: (l_copy(j, i).wait(), 0)[1], 0,
                  unroll=32)
    obuf[s] = heavy + lbuf[s]
    lax.fori_loop(0, BT_H, lambda i, _: (o_copy(j, i).start(), 0)[1], 0,
                  unroll=32)

    @pl.when(j == N_STEPS_H - 1)
    def _():
        lax.fori_loop(0, BT_H, lambda i, _: (o_copy(j - 1, i).wait(), 0)[1],
                      0, unroll=32)
        lax.fori_loop(0, BT_H, lambda i, _: (o_copy(j, i).wait(), 0)[1], 0,
                      unroll=32)


def kernel(x, routing_token, gamma_light, w1, b1, w2, b2,
           gamma_heavy, w3, b3, w4, b4):
    xf = x.reshape(B * N, DIM)
    rt2 = routing_token.reshape(DIM, 1)
    gl = gamma_light.reshape(1, DIM)
    gh = gamma_heavy.reshape(1, DIM)
    b1r = b1.reshape(1, LIGHT_H)
    b2r = b2.reshape(1, DIM)
    b3r = b3.reshape(1, HEAVY_H)
    b4r = b4.reshape(1, DIM)

    sel2d = pl.pallas_call(
        _router_kernel,
        grid=(N_STEPS_L,),
        in_specs=[
            pl.BlockSpec((BT_L, DIM), lambda j: (j, 0)),
            pl.BlockSpec((DIM, 1), lambda j: (0, 0)),
        ],
        out_specs=pl.BlockSpec((1, B * NUM_HEAVY), lambda j: (0, 0)),
        out_shape=jax.ShapeDtypeStruct((1, B * NUM_HEAVY), jnp.int32),
        scratch_shapes=[pltpu.VMEM((N_STEPS_L, BT_L), jnp.float32)],
        compiler_params=pltpu.CompilerParams(
            dimension_semantics=("arbitrary",),
        ),
    )(xf, rt2)

    sel_flat = sel2d.reshape(B * NUM_HEAVY)
    xsel = _get_gather_kernel()(sel_flat, xf)

    lightf = pl.pallas_call(
        _light_kernel,
        grid=(N_STEPS_L,),
        in_specs=[
            pl.BlockSpec((BT_L, DIM), lambda j: (j, 0)),
            pl.BlockSpec((1, DIM), lambda j: (0, 0)),
            pl.BlockSpec((DIM, LIGHT_H), lambda j: (0, 0)),
            pl.BlockSpec((1, LIGHT_H), lambda j: (0, 0)),
            pl.BlockSpec((LIGHT_H, DIM), lambda j: (0, 0)),
            pl.BlockSpec((1, DIM), lambda j: (0, 0)),
        ],
        out_specs=pl.BlockSpec((BT_L, DIM), lambda j: (j, 0)),
        out_shape=jax.ShapeDtypeStruct((B * N, DIM), jnp.float32),
        compiler_params=pltpu.CompilerParams(
            dimension_semantics=("arbitrary",),
        ),
    )(xf, gl, w1, b1r, w2, b2r)

    outf = pl.pallas_call(
        _heavy_kernel,
        grid_spec=pltpu.PrefetchScalarGridSpec(
            num_scalar_prefetch=1,
            grid=(N_STEPS_H,),
            in_specs=[
                pl.BlockSpec((BT_H, DIM), lambda j, sel: (j, 0)),
                pl.BlockSpec(memory_space=pl.MemorySpace.ANY),
                pl.BlockSpec((1, DIM), lambda j, sel: (0, 0)),
                pl.BlockSpec((DIM, HEAVY_H), lambda j, sel: (0, 0)),
                pl.BlockSpec((1, HEAVY_H), lambda j, sel: (0, 0)),
                pl.BlockSpec((HEAVY_H, DIM), lambda j, sel: (0, 0)),
                pl.BlockSpec((1, DIM), lambda j, sel: (0, 0)),
            ],
            out_specs=pl.BlockSpec(memory_space=pl.MemorySpace.ANY),
            scratch_shapes=[
                pltpu.VMEM((2, BT_H, DIM), jnp.float32),
                pltpu.VMEM((2, BT_H, DIM), jnp.float32),
                pltpu.SemaphoreType.DMA((2,)),
                pltpu.SemaphoreType.DMA((2,)),
            ],
        ),
        out_shape=jax.ShapeDtypeStruct((B * N, DIM), jnp.float32),
        input_output_aliases={2: 0},
        compiler_params=pltpu.CompilerParams(
            dimension_semantics=("arbitrary",),
        ),
    )(sel_flat, xsel, lightf, gh, w3, b3r, w4, b4r)

    return outf.reshape(B, N, DIM)


# speedup vs baseline: 1.1271x; 1.1271x over previous
import jax
import jax.numpy as jnp
from jax import lax
from jax.experimental import pallas as pl
from jax.experimental.pallas import tpu as pltpu

B = 2
N = 4096
DIM = 1024
TRASH_SLOT = 2048
NUM_HEAVY = 1024
LIGHT_H = 512
HEAVY_H = 4096
K_CD = 8
EPS_CD = 0.1
N_ITERS = 50

BT_L = 1024
N_STEPS_L = (B * N) // BT_L
BT_H = 256
N_STEPS_H = (B * NUM_HEAVY) // BT_H


def _rmsnorm(x, gamma):
    norm = jnp.sqrt(jnp.sum(x * x, axis=-1, keepdims=True))
    normed = x / jnp.maximum(norm, 1e-12)
    return normed * (DIM ** 0.5) * gamma


def _gelu(x):
    return 0.5 * x * (1.0 + lax.erf(x * (2.0 ** -0.5)))


def _cumsum_lanes(m_f32, nchunk, clen):
    rows = m_f32.reshape(B * nchunk, clen)
    ii = lax.broadcasted_iota(jnp.int32, (clen, clen), 0)
    jj = lax.broadcasted_iota(jnp.int32, (clen, clen), 1)
    tri = (ii <= jj).astype(jnp.float32)
    within = jnp.dot(rows, tri, preferred_element_type=jnp.float32)
    totals = within[:, clen - 1:clen].reshape(B, nchunk)
    ci = lax.broadcasted_iota(jnp.int32, (nchunk, nchunk), 0)
    cj = lax.broadcasted_iota(jnp.int32, (nchunk, nchunk), 1)
    stri = (ci < cj).astype(jnp.float32)
    offs = jnp.dot(totals, stri, preferred_element_type=jnp.float32)
    out = within.reshape(B, nchunk, clen) + offs[:, :, None]
    return out.reshape(B, nchunk * clen)


def _select_slots(scores):
    key = lax.bitcast_convert_type(scores, jnp.int32)
    T = jnp.zeros((B, 1), jnp.int32)
    for bit in range(30, -1, -1):
        cand = T | (1 << bit)
        cnt = jnp.sum((key >= cand).astype(jnp.int32), axis=-1, keepdims=True)
        T = jnp.where(cnt >= NUM_HEAVY, cand, T)
    gt = key > T
    cnt_gt = jnp.sum(gt.astype(jnp.int32), axis=-1, keepdims=True)
    m = NUM_HEAVY - cnt_gt
    eq = key == T
    rank_eq = _cumsum_lanes(eq.astype(jnp.float32), 32, N // 32)
    sel = gt | (eq & (rank_eq <= m.astype(jnp.float32)))
    pos = _cumsum_lanes(sel.astype(jnp.float32), 32, N // 32) - 1.0
    slot = pos.astype(jnp.int32) + (
        lax.broadcasted_iota(jnp.int32, (B, N), 0) * NUM_HEAVY)
    return jnp.where(sel, slot, TRASH_SLOT)


def _compact_matmul(slot):
    slot_flat = slot.reshape(B * N)
    nch = 16
    clen = (B * N) // nch
    acc_hi = jnp.zeros((1, B * NUM_HEAVY), jnp.float32)
    acc_lo = jnp.zeros((1, B * NUM_HEAVY), jnp.float32)
    iota_p = lax.broadcasted_iota(jnp.int32, (clen, B * NUM_HEAVY), 1)
    for c in range(nch):
        sl = slot_flat[c * clen:(c + 1) * clen]
        onehot = (sl[:, None] == iota_p).astype(jnp.float32)
        tok = lax.iota(jnp.int32, clen) + (c * clen)
        tok_hi = (tok // 64).astype(jnp.float32).reshape(1, clen)
        tok_lo = (tok % 64).astype(jnp.float32).reshape(1, clen)
        acc_hi = acc_hi + jnp.dot(tok_hi, onehot,
                                  preferred_element_type=jnp.float32)
        acc_lo = acc_lo + jnp.dot(tok_lo, onehot,
                                  preferred_element_type=jnp.float32)
    return (acc_hi * 64.0 + acc_lo).astype(jnp.int32)


def _light_router_kernel(x_ref, rt_ref, gl_ref, w1_ref, b1_ref, w2_ref, b2_ref,
                         light_ref, sel_ref, s_acc):
    j = pl.program_id(0)
    xb = x_ref[...]
    s_part = jnp.dot(xb, rt_ref[...], preferred_element_type=jnp.float32)
    s_acc[pl.ds(j, 1), :] = s_part.reshape(1, BT_L)
    h = _rmsnorm(xb, gl_ref[...])
    h = jnp.dot(h, w1_ref[...], preferred_element_type=jnp.float32) + b1_ref[...]
    h = _gelu(h)
    light_ref[...] = (
        jnp.dot(h, w2_ref[...], preferred_element_type=jnp.float32) + b2_ref[...]
    )

    @pl.when(j == N_STEPS_L - 1)
    def _():
        s = s_acc[...].reshape(B, N)
        constant = EPS_CD * jnp.log(float(K_CD))
        b = -jnp.maximum(s, 0.0)
        a = jnp.zeros((B, 1), dtype=s.dtype)

        def body(_, carry):
            a, b = carry
            z = (s + b) / EPS_CD
            m = jnp.max(z, axis=-1, keepdims=True)
            lse = jnp.log(jnp.sum(jnp.exp(z - m), axis=-1, keepdims=True)) + m
            a = constant - EPS_CD * lse
            b = -jnp.maximum(s + a, 0.0)
            return a, b

        a, b = lax.fori_loop(0, N_ITERS, body, (a, b))
        scores = jnp.exp((s + a + b) / EPS_CD)
        sel_ref[...] = _compact_matmul(_select_slots(scores))


def _heavy_kernel(sel_ref, x_hbm, _light_alias, gh_ref, w3_ref, b3_ref, w4_ref,
                  b4_ref, out_hbm, xbuf, lbuf, obuf, sem_x, sem_l, sem_o):
    del _light_alias
    j = pl.program_id(0)

    def x_copy(jj, i):
        gidx = sel_ref[jj * BT_H + i]
        s = jax.lax.rem(jj, 2)
        return pltpu.make_async_copy(
            x_hbm.at[pl.ds(gidx, 1), :], xbuf.at[s, pl.ds(i, 1), :],
            sem_x.at[s])

    def l_copy(jj, i):
        gidx = sel_ref[jj * BT_H + i]
        s = jax.lax.rem(jj, 2)
        return pltpu.make_async_copy(
            out_hbm.at[pl.ds(gidx, 1), :], lbuf.at[s, pl.ds(i, 1), :],
            sem_l.at[s])

    def o_copy(jj, i):
        gidx = sel_ref[jj * BT_H + i]
        s = jax.lax.rem(jj, 2)
        return pltpu.make_async_copy(
            obuf.at[s, pl.ds(i, 1), :], out_hbm.at[pl.ds(gidx, 1), :],
            sem_o.at[s])

    def issue_gathers(jj):
        lax.fori_loop(0, BT_H,
                      lambda i, _: (x_copy(jj, i).start(),
                                    l_copy(jj, i).start(), 0)[2], 0,
                      unroll=32)

    @pl.when(j == 0)
    def _():
        issue_gathers(0)

    @pl.when(j + 1 < N_STEPS_H)
    def _():
        issue_gathers(j + 1)

    @pl.when(j >= 2)
    def _():
        lax.fori_loop(0, BT_H, lambda i, _: (o_copy(j - 2, i).wait(), 0)[1],
                      0, unroll=32)

    lax.fori_loop(0, BT_H, lambda i, _: (x_copy(j, i).wait(), 0)[1], 0,
                  unroll=32)
    s = jax.lax.rem(j, 2)
    xb = xbuf[s]
    h = _rmsnorm(xb, gh_ref[...])
    h = jnp.dot(h, w3_ref[...], preferred_element_type=jnp.float32) + b3_ref[...]
    h = _gelu(h)
    heavy = jnp.dot(h, w4_ref[...], preferred_element_type=jnp.float32) + b4_ref[...]

    lax.fori_loop(0, BT_H, lambda i, _: (l_copy(j, i).wait(), 0)[1], 0,
                  unroll=32)
    obuf[s] = heavy + lbuf[s]
    lax.fori_loop(0, BT_H, lambda i, _: (o_copy(j, i).start(), 0)[1], 0,
                  unroll=32)

    @pl.when(j == N_STEPS_H - 1)
    def _():
        lax.fori_loop(0, BT_H, lambda i, _: (o_copy(j - 1, i).wait(), 0)[1],
                      0, unroll=32)
        lax.fori_loop(0, BT_H, lambda i, _: (o_copy(j, i).wait(), 0)[1], 0,
                      unroll=32)


def kernel(x, routing_token, gamma_light, w1, b1, w2, b2,
           gamma_heavy, w3, b3, w4, b4):
    xf = x.reshape(B * N, DIM)
    rt2 = routing_token.reshape(DIM, 1)
    gl = gamma_light.reshape(1, DIM)
    gh = gamma_heavy.reshape(1, DIM)
    b1r = b1.reshape(1, LIGHT_H)
    b2r = b2.reshape(1, DIM)
    b3r = b3.reshape(1, HEAVY_H)
    b4r = b4.reshape(1, DIM)

    lightf, sel2d = pl.pallas_call(
        _light_router_kernel,
        grid=(N_STEPS_L,),
        in_specs=[
            pl.BlockSpec((BT_L, DIM), lambda j: (j, 0)),
            pl.BlockSpec((DIM, 1), lambda j: (0, 0)),
            pl.BlockSpec((1, DIM), lambda j: (0, 0)),
            pl.BlockSpec((DIM, LIGHT_H), lambda j: (0, 0)),
            pl.BlockSpec((1, LIGHT_H), lambda j: (0, 0)),
            pl.BlockSpec((LIGHT_H, DIM), lambda j: (0, 0)),
            pl.BlockSpec((1, DIM), lambda j: (0, 0)),
        ],
        out_specs=[
            pl.BlockSpec((BT_L, DIM), lambda j: (j, 0)),
            pl.BlockSpec((1, B * NUM_HEAVY), lambda j: (0, 0)),
        ],
        out_shape=[
            jax.ShapeDtypeStruct((B * N, DIM), jnp.float32),
            jax.ShapeDtypeStruct((1, B * NUM_HEAVY), jnp.int32),
        ],
        scratch_shapes=[pltpu.VMEM((N_STEPS_L, BT_L), jnp.float32)],
        compiler_params=pltpu.CompilerParams(
            dimension_semantics=("arbitrary",),
        ),
    )(xf, rt2, gl, w1, b1r, w2, b2r)

    sel_flat = sel2d.reshape(B * NUM_HEAVY)

    outf = pl.pallas_call(
        _heavy_kernel,
        grid_spec=pltpu.PrefetchScalarGridSpec(
            num_scalar_prefetch=1,
            grid=(N_STEPS_H,),
            in_specs=[
                pl.BlockSpec(memory_space=pl.MemorySpace.ANY),
                pl.BlockSpec(memory_space=pl.MemorySpace.ANY),
                pl.BlockSpec((1, DIM), lambda j, sel: (0, 0)),
                pl.BlockSpec((DIM, HEAVY_H), lambda j, sel: (0, 0)),
                pl.BlockSpec((1, HEAVY_H), lambda j, sel: (0, 0)),
                pl.BlockSpec((HEAVY_H, DIM), lambda j, sel: (0, 0)),
                pl.BlockSpec((1, DIM), lambda j, sel: (0, 0)),
            ],
            out_specs=pl.BlockSpec(memory_space=pl.MemorySpace.ANY),
            scratch_shapes=[
                pltpu.VMEM((2, BT_H, DIM), jnp.float32),
                pltpu.VMEM((2, BT_H, DIM), jnp.float32),
                pltpu.VMEM((2, BT_H, DIM), jnp.float32),
                pltpu.SemaphoreType.DMA((2,)),
                pltpu.SemaphoreType.DMA((2,)),
                pltpu.SemaphoreType.DMA((2,)),
            ],
        ),
        out_shape=jax.ShapeDtypeStruct((B * N, DIM), jnp.float32),
        input_output_aliases={2: 0},
        compiler_params=pltpu.CompilerParams(
            dimension_semantics=("arbitrary",),
        ),
    )(sel_flat, xf, lightf, gh, w3, b3r, w4, b4r)

    return outf.reshape(B, N, DIM)
